# Initial kernel scaffold; baseline (speedup 1.0000x reference)
import jax
import jax.numpy as jnp
from jax.experimental import pallas as pl


def kernel(x, adj, features, W0, b0, W1, b1, W2, b2, Wout, bout):
    raise NotImplementedError("write your pallas kernel here")


# speedup vs baseline: 2.2910x; 2.2910x over previous
import jax
import jax.numpy as jnp
from jax.experimental import pallas as pl

NBLOCKS = 4


def _dot(a, b):
    return jax.lax.dot_general(a, b, (((1,), (0,)), ((), ())),
                               preferred_element_type=jnp.float32)


def _krylov_body(adj_ref, feat_ref, w0_ref, b0_ref, w1_ref, b1_ref,
                 w2_ref, b2_ref, wout_ref, bout_ref, out_ref):
    A = adj_ref[...]
    nfeat = feat_ref.shape[1]
    nhid = w0_ref.shape[1]

    cur = feat_ref[...]
    acc = _dot(cur, w0_ref[0:nfeat, :])
    for k in range(1, NBLOCKS):
        cur = _dot(A, cur)
        acc = acc + _dot(cur, w0_ref[k * nfeat:(k + 1) * nfeat, :])
    h = jnp.tanh(acc + b0_ref[...])

    for w_ref, b_ref in ((w1_ref, b1_ref), (w2_ref, b2_ref)):
        cur = h
        acc = _dot(cur, w_ref[0:nhid, :])
        for k in range(1, NBLOCKS):
            cur = _dot(A, cur)
            acc = acc + _dot(cur, w_ref[k * nhid:(k + 1) * nhid, :])
        h = jnp.tanh(acc + b_ref[...])

    o = _dot(h, wout_ref[...]) + bout_ref[...]
    nrm = jnp.sqrt(jnp.sum(o * o, axis=1, keepdims=True))
    out_ref[...] = o / jnp.maximum(nrm, 1e-12)


def kernel(x, adj, features, W0, b0, W1, b1, W2, b2, Wout, bout):
    n = adj.shape[0]
    nclass = Wout.shape[1]
    return pl.pallas_call(
        _krylov_body,
        out_shape=jax.ShapeDtypeStruct((n, nclass), jnp.float32),
    )(adj, features, W0, b0.reshape(1, -1), W1, b1.reshape(1, -1),
      W2, b2.reshape(1, -1), Wout, bout.reshape(1, -1))
